# baseline (speedup 1.0000x reference)
import dataclasses
import functools

import jax
import jax.numpy as jnp
from jax import lax
from jax.experimental import pallas as pl
from jax.experimental.pallas import tpu as pltpu
from jax.experimental.pallas import tpu_sc as plsc

N_NODES = 100000
N_PAD = 100096
ROWS = N_PAD // 128
N_EDGES = 6400000
NC, NS, L = 2, 16, 16
NW = NC * NS
CHUNK = 2048
N_GCHUNKS = N_EDGES // CHUNK
NJ = 98
UNROLL = 32
EROWS = N_EDGES // 128
EBLK = 1000
GSTEPS = EROWS // EBLK

REL_EPS = 1e-5
W_L1 = 1.0
W_L1_PHYS = 1.0
W_REL = 0.3
W_SMOOTH = 0.05
W_GC = 0.1

_M_HI = -65536
_M_LO = 65535


def _bf16_hi_bits(x):
    b = lax.bitcast_convert_type(x, jnp.int32)
    lsb = (b >> 16) & 1
    return (b + 0x7FFF + lsb) & _M_HI


def _unpack_hi(v):
    return lax.bitcast_convert_type(v & _M_HI, jnp.float32)


def _unpack_lo(v):
    return lax.bitcast_convert_type(v << 16, jnp.float32)


def _tca_body(p0, p1, p2, t0, t1, t2, px, py,
              c01, c2p, cpos, cfld, nsum):
    preds = [p0[...], p1[...], p2[...]]
    tgts = [t0[...], t1[...], t2[...]]
    sums = []
    errs = []
    for yp, yt in zip(preds, tgts):
        d = yp - yt
        ad = jnp.abs(d)
        sums.append(jnp.sum(jnp.where(ad < 1.0, 0.5 * d * d, ad - 0.5),
                            axis=0, keepdims=True))
        errs.append(d)
    for yp, yt in zip(preds, tgts):
        ad = jnp.abs(yp - yt)
        sums.append(jnp.sum(ad / (jnp.abs(yt) + REL_EPS), axis=0, keepdims=True))
    sums.append(jnp.zeros((2, 128), jnp.float32))
    nsum[...] = jnp.concatenate(sums, axis=0)

    e0, e1, e2 = (_bf16_hi_bits(e) for e in errs)
    potb = _bf16_hi_bits(p0[...])
    exb = _bf16_hi_bits(p1[...])
    eyb = _bf16_hi_bits(p2[...])
    c01[...] = e0 | ((e1 >> 16) & _M_LO)
    c2p[...] = e2 | ((potb >> 16) & _M_LO)
    cfld[...] = exb | ((eyb >> 16) & _M_LO)
    qx = (px[...] * 65536.0).astype(jnp.int32)
    qy = (py[...] * 65536.0).astype(jnp.int32)
    cpos[...] = (qx << 16) | qy


_tca = pl.pallas_call(
    _tca_body,
    out_shape=[
        jax.ShapeDtypeStruct((ROWS, 128), jnp.int32),
        jax.ShapeDtypeStruct((ROWS, 128), jnp.int32),
        jax.ShapeDtypeStruct((ROWS, 128), jnp.int32),
        jax.ShapeDtypeStruct((ROWS, 128), jnp.int32),
        jax.ShapeDtypeStruct((8, 128), jnp.float32),
    ],
)


def _sc_body(e01_h, e2p_h, pos_h, fld_h, ei_h,
             part_h, dp_h, dpx_h, dpy_h, flds_h,
             col, ei0, ei1, bufa0, bufa1, bufb0, bufb1,
             acc0, acc1,
             si0, si1, soa0, soa1, sob0, sob1):
    wid = lax.axis_index("c") * NS + lax.axis_index("s")
    eib = (ei0, ei1)
    bufa = (bufa0, bufa1)
    bufb = (bufb0, bufb1)
    sem_in = (si0, si1)
    sem_out = ((soa0, sob0), (soa1, sob1))

    def run_pass(col_hbm, body_maker, n_carry, acc_dsts, out_hbms):
        pltpu.sync_copy(col_hbm, col)

        def in_cp(cid, b):
            return pltpu.make_async_copy(
                ei_h.at[:, pl.ds(cid * CHUNK, CHUNK)], eib[b], sem_in[b])

        def out_cps(cid, b):
            base = cid * CHUNK
            cps = []
            for buf, hbm, sem in zip((bufa[b], bufb[b]), out_hbms, sem_out[b]):
                if hbm is not None:
                    cps.append(pltpu.make_async_copy(
                        buf, hbm.at[pl.ds(base, CHUNK)], sem))
            return cps

        n_out = len([h for h in out_hbms if h is not None])

        def compute(b):
            if n_carry:
                init = tuple(jnp.zeros((L,), jnp.float32)
                             for _ in range(n_carry))
                res = plsc.parallel_loop(
                    0, CHUNK, step=L, unroll=UNROLL,
                    carry=init)(body_maker(b))
                for aref, val in zip(acc_dsts, jax.tree.leaves(res)):
                    aref[...] = aref[...] + val
            else:
                plsc.parallel_loop(
                    0, CHUNK, step=L, unroll=UNROLL)(body_maker(b))

        in_cp(wid, 0).start()

        @pl.loop(0, NJ, step=2)
        def _chunks(j):
            cid0 = wid + NW * j
            cid1 = wid + NW * (j + 1)
            @pl.when(cid1 < N_GCHUNKS)
            def _():
                in_cp(cid1, 1).start()
            in_cp(cid0, 0).wait()
            if n_out:
                @pl.when(j >= 2)
                def _():
                    for c in out_cps(cid0, 0):
                        c.wait()
            compute(0)
            if n_out:
                for c in out_cps(cid0, 0):
                    c.start()

            @pl.when(cid1 < N_GCHUNKS)
            def _():
                cid2 = wid + NW * (j + 2)

                @pl.when(cid2 < N_GCHUNKS)
                def _():
                    in_cp(cid2, 0).start()
                in_cp(cid1, 1).wait()
                if n_out:
                    @pl.when(j >= 2)
                    def _():
                        for c in out_cps(cid1, 1):
                            c.wait()
                compute(1)
                if n_out:
                    for c in out_cps(cid1, 1):
                        c.start()

        if n_out:
            for b in (0, 1):
                for c in out_cps(wid, b):
                    c.wait()

    acc0[...] = jnp.zeros((L,), jnp.float32)
    acc1[...] = jnp.zeros((L,), jnp.float32)

    def p1_body(b):
        def body(g, c):
            a0, a1 = c
            vs = plsc.load_gather(col, [eib[b][0, pl.ds(g, L)]])
            vd = plsc.load_gather(col, [eib[b][1, pl.ds(g, L)]])
            a0 = a0 + jnp.abs(_unpack_hi(vs) - _unpack_hi(vd))
            a1 = a1 + jnp.abs(_unpack_lo(vs) - _unpack_lo(vd))
            return (a0, a1)
        return body

    run_pass(e01_h, p1_body, 2, (acc0, acc1), (None, None))
    pltpu.sync_copy(acc0, part_h.at[0, wid])
    pltpu.sync_copy(acc1, part_h.at[1, wid])

    acc0[...] = jnp.zeros((L,), jnp.float32)

    def p2_body(b):
        def body(g, c):
            (a0,) = c
            vs = plsc.load_gather(col, [eib[b][0, pl.ds(g, L)]])
            vd = plsc.load_gather(col, [eib[b][1, pl.ds(g, L)]])
            a0 = a0 + jnp.abs(_unpack_hi(vs) - _unpack_hi(vd))
            bufa[b][pl.ds(g, L)] = _unpack_lo(vd) - _unpack_lo(vs)
            return (a0,)
        return body

    run_pass(e2p_h, p2_body, 1, (acc0,), (dp_h, None))
    pltpu.sync_copy(acc0, part_h.at[2, wid])

    def p3_body(b):
        def body(g):
            vs = plsc.load_gather(col, [eib[b][0, pl.ds(g, L)]])
            vd = plsc.load_gather(col, [eib[b][1, pl.ds(g, L)]])
            dqx = ((vd >> 16) & _M_LO) - ((vs >> 16) & _M_LO)
            dqy = (vd & _M_LO) - (vs & _M_LO)
            bufa[b][pl.ds(g, L)] = dqx.astype(jnp.float32) * (1.0 / 65536.0)
            bufb[b][pl.ds(g, L)] = dqy.astype(jnp.float32) * (1.0 / 65536.0)
        return body

    run_pass(pos_h, p3_body, 0, (), (dpx_h, dpy_h))

    def p4_body(b):
        def body(g):
            vs = plsc.load_gather(col, [eib[b][0, pl.ds(g, L)]])
            bufa[b][pl.ds(g, L)] = lax.bitcast_convert_type(vs, jnp.float32)
        return body

    run_pass(fld_h, p4_body, 0, (), (flds_h, None))


_sc_cp = pltpu.CompilerParams()
if "needs_layout_passes" in pltpu.CompilerParams.__dataclass_fields__:
    _sc_cp = dataclasses.replace(_sc_cp, needs_layout_passes=False)

_sc = functools.partial(
    pl.kernel,
    mesh=plsc.VectorSubcoreMesh(core_axis_name="c", subcore_axis_name="s"),
    compiler_params=_sc_cp,
    out_type=[
        jax.ShapeDtypeStruct((3, NW, L), jnp.float32),
        jax.ShapeDtypeStruct((N_EDGES,), jnp.float32),
        jax.ShapeDtypeStruct((N_EDGES,), jnp.float32),
        jax.ShapeDtypeStruct((N_EDGES,), jnp.float32),
        jax.ShapeDtypeStruct((N_EDGES,), jnp.float32),
    ],
    scratch_types=(
        [pltpu.VMEM((N_PAD,), jnp.int32)]
        + [pltpu.VMEM((2, CHUNK), jnp.int32)] * 2
        + [pltpu.VMEM((CHUNK,), jnp.float32)] * 4
        + [pltpu.VMEM((L,), jnp.float32)] * 2
        + [pltpu.SemaphoreType.DMA] * 6
    ),
)(_sc_body)


def _tcb_body(dp, dpx, dpy, flds, part, nsum, out, acc):
    i = pl.program_id(0)

    @pl.when(i == 0)
    def _():
        acc[0] = 0.0

    x = dpx[...]
    y = dpy[...]
    ds = jnp.maximum(jnp.sqrt(x * x + y * y), 1e-6)
    v = lax.bitcast_convert_type(flds[...], jnp.int32)
    exs = _unpack_hi(v)
    eys = _unpack_lo(v)
    d = (exs * x + eys * y + dp[...]) / ds
    ad = jnp.abs(d)
    blk = jnp.sum(jnp.where(ad < 1.0, 0.5 * d * d, ad - 0.5))
    acc[0] = acc[0] + blk

    @pl.when(i == GSTEPS - 1)
    def _():
        tv = jnp.sum(part[...], axis=1)
        ns = jnp.sum(nsum[...], axis=1)
        inv_n = 1.0 / N_NODES
        inv_e = 1.0 / N_EDGES
        total = jnp.float32(0.0)
        for ch in range(3):
            total = total + (W_L1 + W_L1_PHYS) * ns[ch] * inv_n
            total = total + W_REL * ns[3 + ch] * inv_n
            total = total + W_SMOOTH * tv[ch] * inv_e
        total = total + W_GC * acc[0] * inv_e
        out[...] = jnp.reshape(total, (1, 1))


_tcb = pl.pallas_call(
    _tcb_body,
    grid=(GSTEPS,),
    in_specs=[
        pl.BlockSpec((EBLK, 128), lambda i: (i, 0)),
        pl.BlockSpec((EBLK, 128), lambda i: (i, 0)),
        pl.BlockSpec((EBLK, 128), lambda i: (i, 0)),
        pl.BlockSpec((EBLK, 128), lambda i: (i, 0)),
        pl.BlockSpec((3, NW * L), lambda i: (0, 0)),
        pl.BlockSpec((8, 128), lambda i: (0, 0)),
    ],
    out_specs=pl.BlockSpec((1, 1), lambda i: (0, 0)),
    out_shape=jax.ShapeDtypeStruct((1, 1), jnp.float32),
    scratch_shapes=[pltpu.SMEM((1,), jnp.float32)],
)


def _prep_node(x):
    x = x.reshape(-1).astype(jnp.float32)
    return jnp.pad(x, (0, N_PAD - N_NODES)).reshape(ROWS, 128)


def kernel(pred_ElectrostaticPotential, pred_ElectricField_x,
           pred_ElectricField_y, target_tensor, pos, edge_index):
    p0 = _prep_node(pred_ElectrostaticPotential)
    p1 = _prep_node(pred_ElectricField_x)
    p2 = _prep_node(pred_ElectricField_y)
    t0 = _prep_node(target_tensor[:, 0])
    t1 = _prep_node(target_tensor[:, 1])
    t2 = _prep_node(target_tensor[:, 2])
    px = _prep_node(pos[:, 0])
    py = _prep_node(pos[:, 1])

    c01, c2p, cpos, cfld, nsum = _tca(p0, p1, p2, t0, t1, t2, px, py)

    part, dp, dpx, dpy, flds = _sc(
        c01.reshape(-1), c2p.reshape(-1), cpos.reshape(-1), cfld.reshape(-1),
        edge_index.astype(jnp.int32))

    out = _tcb(dp.reshape(EROWS, 128), dpx.reshape(EROWS, 128),
               dpy.reshape(EROWS, 128), flds.reshape(EROWS, 128),
               part.reshape(3, NW * L), nsum)
    return out[0, 0]


# speedup vs baseline: 1.1632x; 1.1632x over previous
import dataclasses
import functools

import jax
import jax.numpy as jnp
from jax import lax
from jax.experimental import pallas as pl
from jax.experimental.pallas import tpu as pltpu
from jax.experimental.pallas import tpu_sc as plsc

N_NODES = 100000
N_PAD = 100096
ROWS = N_PAD // 128
N_EDGES = 6400000
NC, NS, L = 2, 16, 16
NW = NC * NS
CHUNK = 2048
N_GCHUNKS = N_EDGES // CHUNK
NJ = 98
UNROLL = 16
EROWS = N_EDGES // 128
EBLK = 1000
GSTEPS = EROWS // EBLK

REL_EPS = 1e-5
W_L1 = 1.0
W_L1_PHYS = 1.0
W_REL = 0.3
W_SMOOTH = 0.05
W_GC = 0.1

_M_HI = -65536
_M_LO = 65535


def _bf16_hi_bits(x):
    b = lax.bitcast_convert_type(x, jnp.int32)
    lsb = (b >> 16) & 1
    return (b + 0x7FFF + lsb) & _M_HI


def _unpack_hi(v):
    return lax.bitcast_convert_type(v & _M_HI, jnp.float32)


def _unpack_lo(v):
    return lax.bitcast_convert_type(v << 16, jnp.float32)


def _tca_body(p0, p1, p2, t0, t1, t2, px, py,
              c01, c2p, cpos, cfld, nsum):
    preds = [p0[...], p1[...], p2[...]]
    tgts = [t0[...], t1[...], t2[...]]
    sums = []
    errs = []
    for yp, yt in zip(preds, tgts):
        d = yp - yt
        ad = jnp.abs(d)
        sums.append(jnp.sum(jnp.where(ad < 1.0, 0.5 * d * d, ad - 0.5),
                            axis=0, keepdims=True))
        errs.append(d)
    for yp, yt in zip(preds, tgts):
        ad = jnp.abs(yp - yt)
        sums.append(jnp.sum(ad / (jnp.abs(yt) + REL_EPS), axis=0, keepdims=True))
    sums.append(jnp.zeros((2, 128), jnp.float32))
    nsum[...] = jnp.concatenate(sums, axis=0)

    e0, e1, e2 = (_bf16_hi_bits(e) for e in errs)
    potb = _bf16_hi_bits(p0[...])
    exb = _bf16_hi_bits(p1[...])
    eyb = _bf16_hi_bits(p2[...])
    c01[...] = e0 | ((e1 >> 16) & _M_LO)
    c2p[...] = e2 | ((potb >> 16) & _M_LO)
    cfld[...] = exb | ((eyb >> 16) & _M_LO)
    qx = (px[...] * 65536.0).astype(jnp.int32)
    qy = (py[...] * 65536.0).astype(jnp.int32)
    cpos[...] = (qx << 16) | qy


_tca = pl.pallas_call(
    _tca_body,
    out_shape=[
        jax.ShapeDtypeStruct((ROWS, 128), jnp.int32),
        jax.ShapeDtypeStruct((ROWS, 128), jnp.int32),
        jax.ShapeDtypeStruct((ROWS, 128), jnp.int32),
        jax.ShapeDtypeStruct((ROWS, 128), jnp.int32),
        jax.ShapeDtypeStruct((8, 128), jnp.float32),
    ],
)


def _sc_body(e01_h, e2p_h, pos_h, fld_h, ei_h,
             part_h, dp_h, dpx_h, dpy_h, flds_h,
             col, ei0, ei1, bufa0, bufa1, bufb0, bufb1,
             acc0, acc1,
             si0, si1, soa0, soa1, sob0, sob1):
    wid = lax.axis_index("c") * NS + lax.axis_index("s")
    eib = (ei0, ei1)
    bufa = (bufa0, bufa1)
    bufb = (bufb0, bufb1)
    sem_in = (si0, si1)
    sem_out = ((soa0, sob0), (soa1, sob1))

    def run_pass(col_hbm, body_maker, n_carry, acc_dsts, out_hbms):
        pltpu.sync_copy(col_hbm, col)

        def in_cp(cid, b):
            return pltpu.make_async_copy(
                ei_h.at[:, pl.ds(cid * CHUNK, CHUNK)], eib[b], sem_in[b])

        def out_cps(cid, b):
            base = cid * CHUNK
            cps = []
            for buf, hbm, sem in zip((bufa[b], bufb[b]), out_hbms, sem_out[b]):
                if hbm is not None:
                    cps.append(pltpu.make_async_copy(
                        buf, hbm.at[pl.ds(base, CHUNK)], sem))
            return cps

        n_out = len([h for h in out_hbms if h is not None])

        def compute(b):
            if n_carry:
                init = tuple(jnp.zeros((L,), jnp.float32)
                             for _ in range(n_carry))
                res = plsc.parallel_loop(
                    0, CHUNK, step=L, unroll=UNROLL,
                    carry=init)(body_maker(b))
                for aref, val in zip(acc_dsts, jax.tree.leaves(res)):
                    aref[...] = aref[...] + val
            else:
                plsc.parallel_loop(
                    0, CHUNK, step=L, unroll=UNROLL)(body_maker(b))

        in_cp(wid, 0).start()

        @pl.loop(0, NJ, step=2)
        def _chunks(j):
            cid0 = wid + NW * j
            cid1 = wid + NW * (j + 1)
            @pl.when(cid1 < N_GCHUNKS)
            def _():
                in_cp(cid1, 1).start()
            in_cp(cid0, 0).wait()
            if n_out:
                @pl.when(j >= 2)
                def _():
                    for c in out_cps(cid0, 0):
                        c.wait()
            compute(0)
            if n_out:
                for c in out_cps(cid0, 0):
                    c.start()

            @pl.when(cid1 < N_GCHUNKS)
            def _():
                cid2 = wid + NW * (j + 2)

                @pl.when(cid2 < N_GCHUNKS)
                def _():
                    in_cp(cid2, 0).start()
                in_cp(cid1, 1).wait()
                if n_out:
                    @pl.when(j >= 2)
                    def _():
                        for c in out_cps(cid1, 1):
                            c.wait()
                compute(1)
                if n_out:
                    for c in out_cps(cid1, 1):
                        c.start()

        if n_out:
            for b in (0, 1):
                for c in out_cps(wid, b):
                    c.wait()

    acc0[...] = jnp.zeros((L,), jnp.float32)
    acc1[...] = jnp.zeros((L,), jnp.float32)

    def p1_body(b):
        def body(g, c):
            a0, a1 = c
            vs = plsc.load_gather(col, [eib[b][0, pl.ds(g, L)]])
            vd = plsc.load_gather(col, [eib[b][1, pl.ds(g, L)]])
            a0 = a0 + jnp.abs(_unpack_hi(vs) - _unpack_hi(vd))
            a1 = a1 + jnp.abs(_unpack_lo(vs) - _unpack_lo(vd))
            return (a0, a1)
        return body

    run_pass(e01_h, p1_body, 2, (acc0, acc1), (None, None))
    pltpu.sync_copy(acc0, part_h.at[0, wid])
    pltpu.sync_copy(acc1, part_h.at[1, wid])

    acc0[...] = jnp.zeros((L,), jnp.float32)

    def p2_body(b):
        def body(g, c):
            (a0,) = c
            vs = plsc.load_gather(col, [eib[b][0, pl.ds(g, L)]])
            vd = plsc.load_gather(col, [eib[b][1, pl.ds(g, L)]])
            a0 = a0 + jnp.abs(_unpack_hi(vs) - _unpack_hi(vd))
            bufa[b][pl.ds(g, L)] = _unpack_lo(vd) - _unpack_lo(vs)
            return (a0,)
        return body

    run_pass(e2p_h, p2_body, 1, (acc0,), (dp_h, None))
    pltpu.sync_copy(acc0, part_h.at[2, wid])

    def p3_body(b):
        def body(g):
            vs = plsc.load_gather(col, [eib[b][0, pl.ds(g, L)]])
            vd = plsc.load_gather(col, [eib[b][1, pl.ds(g, L)]])
            dqx = ((vd >> 16) & _M_LO) - ((vs >> 16) & _M_LO)
            dqy = (vd & _M_LO) - (vs & _M_LO)
            bufa[b][pl.ds(g, L)] = dqx.astype(jnp.float32) * (1.0 / 65536.0)
            bufb[b][pl.ds(g, L)] = dqy.astype(jnp.float32) * (1.0 / 65536.0)
        return body

    run_pass(pos_h, p3_body, 0, (), (dpx_h, dpy_h))

    def p4_body(b):
        def body(g):
            vs = plsc.load_gather(col, [eib[b][0, pl.ds(g, L)]])
            bufa[b][pl.ds(g, L)] = lax.bitcast_convert_type(vs, jnp.float32)
        return body

    run_pass(fld_h, p4_body, 0, (), (flds_h, None))


_sc_cp = pltpu.CompilerParams()
if "needs_layout_passes" in pltpu.CompilerParams.__dataclass_fields__:
    _sc_cp = dataclasses.replace(_sc_cp, needs_layout_passes=False)

_sc = functools.partial(
    pl.kernel,
    mesh=plsc.VectorSubcoreMesh(core_axis_name="c", subcore_axis_name="s"),
    compiler_params=_sc_cp,
    out_type=[
        jax.ShapeDtypeStruct((3, NW, L), jnp.float32),
        jax.ShapeDtypeStruct((N_EDGES,), jnp.float32),
        jax.ShapeDtypeStruct((N_EDGES,), jnp.float32),
        jax.ShapeDtypeStruct((N_EDGES,), jnp.float32),
        jax.ShapeDtypeStruct((N_EDGES,), jnp.float32),
    ],
    scratch_types=(
        [pltpu.VMEM((N_PAD,), jnp.int32)]
        + [pltpu.VMEM((2, CHUNK), jnp.int32)] * 2
        + [pltpu.VMEM((CHUNK,), jnp.float32)] * 4
        + [pltpu.VMEM((L,), jnp.float32)] * 2
        + [pltpu.SemaphoreType.DMA] * 6
    ),
)(_sc_body)


def _tcb_body(dp, dpx, dpy, flds, part, nsum, out, acc):
    i = pl.program_id(0)

    @pl.when(i == 0)
    def _():
        acc[0] = 0.0

    x = dpx[...]
    y = dpy[...]
    ds = jnp.maximum(jnp.sqrt(x * x + y * y), 1e-6)
    v = lax.bitcast_convert_type(flds[...], jnp.int32)
    exs = _unpack_hi(v)
    eys = _unpack_lo(v)
    d = (exs * x + eys * y + dp[...]) / ds
    ad = jnp.abs(d)
    blk = jnp.sum(jnp.where(ad < 1.0, 0.5 * d * d, ad - 0.5))
    acc[0] = acc[0] + blk

    @pl.when(i == GSTEPS - 1)
    def _():
        tv = jnp.sum(part[...], axis=1)
        ns = jnp.sum(nsum[...], axis=1)
        inv_n = 1.0 / N_NODES
        inv_e = 1.0 / N_EDGES
        total = jnp.float32(0.0)
        for ch in range(3):
            total = total + (W_L1 + W_L1_PHYS) * ns[ch] * inv_n
            total = total + W_REL * ns[3 + ch] * inv_n
            total = total + W_SMOOTH * tv[ch] * inv_e
        total = total + W_GC * acc[0] * inv_e
        out[...] = jnp.reshape(total, (1, 1))


_tcb = pl.pallas_call(
    _tcb_body,
    grid=(GSTEPS,),
    in_specs=[
        pl.BlockSpec((EBLK, 128), lambda i: (i, 0)),
        pl.BlockSpec((EBLK, 128), lambda i: (i, 0)),
        pl.BlockSpec((EBLK, 128), lambda i: (i, 0)),
        pl.BlockSpec((EBLK, 128), lambda i: (i, 0)),
        pl.BlockSpec((3, NW * L), lambda i: (0, 0)),
        pl.BlockSpec((8, 128), lambda i: (0, 0)),
    ],
    out_specs=pl.BlockSpec((1, 1), lambda i: (0, 0)),
    out_shape=jax.ShapeDtypeStruct((1, 1), jnp.float32),
    scratch_shapes=[pltpu.SMEM((1,), jnp.float32)],
)


def _prep_node(x):
    x = x.reshape(-1).astype(jnp.float32)
    return jnp.pad(x, (0, N_PAD - N_NODES)).reshape(ROWS, 128)


def kernel(pred_ElectrostaticPotential, pred_ElectricField_x,
           pred_ElectricField_y, target_tensor, pos, edge_index):
    p0 = _prep_node(pred_ElectrostaticPotential)
    p1 = _prep_node(pred_ElectricField_x)
    p2 = _prep_node(pred_ElectricField_y)
    t0 = _prep_node(target_tensor[:, 0])
    t1 = _prep_node(target_tensor[:, 1])
    t2 = _prep_node(target_tensor[:, 2])
    px = _prep_node(pos[:, 0])
    py = _prep_node(pos[:, 1])

    c01, c2p, cpos, cfld, nsum = _tca(p0, p1, p2, t0, t1, t2, px, py)

    part, dp, dpx, dpy, flds = _sc(
        c01.reshape(-1), c2p.reshape(-1), cpos.reshape(-1), cfld.reshape(-1),
        edge_index.astype(jnp.int32))

    out = _tcb(dp.reshape(EROWS, 128), dpx.reshape(EROWS, 128),
               dpy.reshape(EROWS, 128), flds.reshape(EROWS, 128),
               part.reshape(3, NW * L), nsum)
    return out[0, 0]
